# Initial kernel scaffold; baseline (speedup 1.0000x reference)
import jax
import jax.numpy as jnp
from jax.experimental import pallas as pl


def kernel(x, edge_index, adj, ADJ1, W_in, b_in, W1, W2, Wf, W_out, b_out):
    raise NotImplementedError("write your pallas kernel here")


# speedup vs baseline: 3.8227x; 3.8227x over previous
import dataclasses
import functools

import jax
import jax.numpy as jnp
from jax import lax
from jax.experimental import pallas as pl
from jax.experimental.pallas import tpu as pltpu
from jax.experimental.pallas import tpu_sc as plsc

N = 10000
E = 320000
FEAT = 128
H = 128
C = 40

_F32 = jnp.float32
_sds = jax.ShapeDtypeStruct


_ROWS = 2000
_NB = N // _ROWS


def _dotT(a, b):
    return lax.dot_general(a, b, (((1,), (1,)), ((), ())),
                           preferred_element_type=_F32,
                           precision=lax.Precision.HIGHEST)


def _tc0_body(x_ref, win_ref, bin_ref, w1_ref, w2_ref, wf_ref,
              h3_ref, h4_ref, h1_ref):
    q = _dotT(x_ref[...], win_ref[...]) + bin_ref[...]
    h3_ref[...] = _dotT(q, w1_ref[...])
    h4_ref[...] = _dotT(q, w2_ref[...])
    h1_ref[...] = jnp.maximum(_dotT(q, wf_ref[...]), 0.0)


def _norm_rows(xo):
    nrm = jnp.maximum(jnp.sqrt(jnp.sum(xo * xo, axis=1, keepdims=True)), 1e-12)
    return xo / nrm


def _tc1_body(h1_ref, p0_ref, p1_ref, w1_ref, w2_ref, wf_ref,
              h3_ref, h4_ref, h1o_ref):
    q = _norm_rows(h1_ref[...] + p0_ref[...] + p1_ref[...])
    h3_ref[...] = _dotT(q, w1_ref[...])
    h4_ref[...] = _dotT(q, w2_ref[...])
    h1o_ref[...] = jnp.maximum(_dotT(q, wf_ref[...]), 0.0)


def _tc2_body(h1_ref, p0_ref, p1_ref, wout_ref, bout_ref, o_ref):
    q = _norm_rows(h1_ref[...] + p0_ref[...] + p1_ref[...])
    logits = _dotT(q, wout_ref[...]) + bout_ref[...]
    m = jnp.max(logits, axis=1, keepdims=True)
    z = logits - m
    lse = jnp.log(jnp.sum(jnp.exp(z), axis=1, keepdims=True))
    o_ref[...] = z - lse


_row_spec = pl.BlockSpec((_ROWS, H), lambda i: (i, 0))
_full_spec = pl.BlockSpec((H, H), lambda i: (0, 0))
_bias_spec = pl.BlockSpec((1, H), lambda i: (0, 0))

_tc0 = pl.pallas_call(
    _tc0_body,
    grid=(_NB,),
    in_specs=[pl.BlockSpec((_ROWS, FEAT), lambda i: (i, 0)),
              pl.BlockSpec((H, FEAT), lambda i: (0, 0)),
              _bias_spec, _full_spec, _full_spec, _full_spec],
    out_specs=[_row_spec, _row_spec, _row_spec],
    out_shape=[_sds((N, H), _F32)] * 3,
)

_tc1 = pl.pallas_call(
    _tc1_body,
    grid=(_NB,),
    in_specs=[_row_spec, _row_spec, _row_spec,
              _full_spec, _full_spec, _full_spec],
    out_specs=[_row_spec, _row_spec, _row_spec],
    out_shape=[_sds((N, H), _F32)] * 3,
)

_tc2 = pl.pallas_call(
    _tc2_body,
    grid=(_NB,),
    in_specs=[_row_spec, _row_spec, _row_spec,
              pl.BlockSpec((C, H), lambda i: (0, 0)),
              pl.BlockSpec((1, C), lambda i: (0, 0))],
    out_specs=pl.BlockSpec((_ROWS, C), lambda i: (i, 0)),
    out_shape=_sds((N, C), _F32),
)


_NC = 2
_NS = 16
_NW = _NC * _NS
_EPW = E // _NW
_BC = 80
_NCH = _EPW // _BC
_DR = 624
_TAIL = N - _NS * _DR
_ZR = 16
_LC = H // 16

_mesh = plsc.VectorSubcoreMesh(core_axis_name="c", subcore_axis_name="s")

_sc_cp = pltpu.CompilerParams()
if "needs_layout_passes" in pltpu.CompilerParams.__dataclass_fields__:
    _sc_cp = dataclasses.replace(_sc_cp, needs_layout_passes=False)


@functools.partial(
    pl.kernel,
    mesh=_mesh,
    compiler_params=_sc_cp,
    out_type=_sds((_NC * N, H), _F32),
    scratch_types=[
        pltpu.VMEM((_BC,), jnp.int32),
        pltpu.VMEM((_BC,), jnp.int32),
        pltpu.VMEM((_BC, H), _F32),
        pltpu.VMEM((_BC, H), _F32),
        pltpu.VMEM((_BC, H), _F32),
        pltpu.VMEM((_ZR, H), _F32),
        pltpu.VMEM_SHARED((N, H), _F32),
        pltpu.SemaphoreType.DMA,
    ],
)
def _sc_edge(h3_hbm, h4_hbm, row_hbm, col_hbm, out_hbm,
             rowi, coli, h3r, h4r, h4c, zbuf, acc, sem):
    ci = lax.axis_index("c")
    si = lax.axis_index("s")

    @pl.loop(0, _ZR)
    def _(i):
        for j in range(_LC):
            zbuf[i, pl.ds(j * 16, 16)] = jnp.zeros((16,), _F32)

    @pl.loop(0, _DR // _ZR)
    def _(t):
        off = pl.multiple_of(si * _DR + t * _ZR, 8)
        pltpu.sync_copy(zbuf, acc.at[pl.ds(off, _ZR)])

    @pl.when(si == _NS - 1)
    def _():
        pltpu.sync_copy(zbuf.at[pl.ds(0, _TAIL)],
                        acc.at[pl.ds(_NS * _DR, _TAIL)])

    plsc.subcore_barrier()

    @pl.loop(0, _NCH)
    def _(c):
        eb = (ci * _NS + si) * _EPW + c * _BC
        pltpu.sync_copy(row_hbm.at[pl.ds(eb, _BC)], rowi)
        pltpu.sync_copy(col_hbm.at[pl.ds(eb, _BC)], coli)
        g1 = pltpu.async_copy(h3_hbm.at[rowi], h3r, sem)
        g2 = pltpu.async_copy(h4_hbm.at[rowi], h4r, sem)
        g3 = pltpu.async_copy(h4_hbm.at[coli], h4c, sem)
        g1.wait()
        g2.wait()
        g3.wait()

        @pl.loop(0, _BC)
        def _(e):
            h3v = [h3r[e, pl.ds(16 * j, 16)] for j in range(_LC)]
            h4cv = [h4c[e, pl.ds(16 * j, 16)] for j in range(_LC)]
            accv = h3v[0] * h4cv[0]
            for j in range(1, _LC):
                accv = accv + h3v[j] * h4cv[j]
            dvec = jnp.full((16,), jnp.sum(accv), _F32)
            svec = 1.0 / (1.0 + jnp.exp(dvec))
            for j in range(_LC):
                h4v = h4r[e, pl.ds(16 * j, 16)]
                h4c[e, pl.ds(16 * j, 16)] = svec * (h3v[j] + h4v) - h4v

        pltpu.sync_copy(h4c, acc.at[coli], add=True)

    plsc.subcore_barrier()

    doff = pl.multiple_of(si * _DR, 8)
    ooff = pl.multiple_of(ci * N + si * _DR, 8)
    pltpu.sync_copy(acc.at[pl.ds(doff, _DR)], out_hbm.at[pl.ds(ooff, _DR)])

    @pl.when(si == _NS - 1)
    def _():
        toff = pl.multiple_of(ci * N + _NS * _DR, 8)
        pltpu.sync_copy(acc.at[pl.ds(_NS * _DR, _TAIL)],
                        out_hbm.at[pl.ds(toff, _TAIL)])


def kernel(x, edge_index, adj, ADJ1, W_in, b_in, W1, W2, Wf, W_out, b_out):
    row = edge_index[0].astype(jnp.int32)
    col = edge_index[1].astype(jnp.int32)
    b_in2 = b_in.reshape(1, H)
    b_out2 = b_out.reshape(1, C)

    h3, h4, h1 = _tc0(x, W_in, b_in2, W1[0], W2[0], Wf[0])
    p = _sc_edge(h3, h4, row, col)
    h3, h4, h1 = _tc1(h1, p[:N], p[N:], W1[1], W2[1], Wf[1])
    p = _sc_edge(h3, h4, row, col)
    return _tc2(h1, p[:N], p[N:], W_out, b_out2)


# speedup vs baseline: 4.0787x; 1.0670x over previous
import dataclasses
import functools

import jax
import jax.numpy as jnp
from jax import lax
from jax.experimental import pallas as pl
from jax.experimental.pallas import tpu as pltpu
from jax.experimental.pallas import tpu_sc as plsc

N = 10000
E = 320000
FEAT = 128
H = 128
C = 40

_F32 = jnp.float32
_sds = jax.ShapeDtypeStruct


_ROWS = 2000
_NB = N // _ROWS


def _dotT(a, b):
    return lax.dot_general(a, b, (((1,), (1,)), ((), ())),
                           preferred_element_type=_F32,
                           precision=lax.Precision.HIGHEST)


def _tc0_body(x_ref, win_ref, bin_ref, w1_ref, w2_ref, wf_ref,
              h3_ref, h4_ref, h1_ref):
    q = _dotT(x_ref[...], win_ref[...]) + bin_ref[...]
    h3_ref[...] = _dotT(q, w1_ref[...])
    h4_ref[...] = _dotT(q, w2_ref[...])
    h1_ref[...] = jnp.maximum(_dotT(q, wf_ref[...]), 0.0)


def _norm_rows(xo):
    nrm = jnp.maximum(jnp.sqrt(jnp.sum(xo * xo, axis=1, keepdims=True)), 1e-12)
    return xo / nrm


def _tc1_body(h1_ref, p0_ref, p1_ref, w1_ref, w2_ref, wf_ref,
              h3_ref, h4_ref, h1o_ref):
    q = _norm_rows(h1_ref[...] + p0_ref[...] + p1_ref[...])
    h3_ref[...] = _dotT(q, w1_ref[...])
    h4_ref[...] = _dotT(q, w2_ref[...])
    h1o_ref[...] = jnp.maximum(_dotT(q, wf_ref[...]), 0.0)


def _tc2_body(h1_ref, p0_ref, p1_ref, wout_ref, bout_ref, o_ref):
    q = _norm_rows(h1_ref[...] + p0_ref[...] + p1_ref[...])
    logits = _dotT(q, wout_ref[...]) + bout_ref[...]
    m = jnp.max(logits, axis=1, keepdims=True)
    z = logits - m
    lse = jnp.log(jnp.sum(jnp.exp(z), axis=1, keepdims=True))
    o_ref[...] = z - lse


_row_spec = pl.BlockSpec((_ROWS, H), lambda i: (i, 0))
_full_spec = pl.BlockSpec((H, H), lambda i: (0, 0))
_bias_spec = pl.BlockSpec((1, H), lambda i: (0, 0))

_tc0 = pl.pallas_call(
    _tc0_body,
    grid=(_NB,),
    in_specs=[pl.BlockSpec((_ROWS, FEAT), lambda i: (i, 0)),
              pl.BlockSpec((H, FEAT), lambda i: (0, 0)),
              _bias_spec, _full_spec, _full_spec, _full_spec],
    out_specs=[_row_spec, _row_spec, _row_spec],
    out_shape=[_sds((N, H), _F32)] * 3,
)

_tc1 = pl.pallas_call(
    _tc1_body,
    grid=(_NB,),
    in_specs=[_row_spec, _row_spec, _row_spec,
              _full_spec, _full_spec, _full_spec],
    out_specs=[_row_spec, _row_spec, _row_spec],
    out_shape=[_sds((N, H), _F32)] * 3,
)

_tc2 = pl.pallas_call(
    _tc2_body,
    grid=(_NB,),
    in_specs=[_row_spec, _row_spec, _row_spec,
              pl.BlockSpec((C, H), lambda i: (0, 0)),
              pl.BlockSpec((1, C), lambda i: (0, 0))],
    out_specs=pl.BlockSpec((_ROWS, C), lambda i: (i, 0)),
    out_shape=_sds((N, C), _F32),
)


_NC = 2
_NS = 16
_NW = _NC * _NS
_EPW = E // _NW
_BC = 80
_NCH = _EPW // _BC
_DR = 624
_TAIL = N - _NS * _DR
_ZR = 16
_LC = H // 16

_mesh = plsc.VectorSubcoreMesh(core_axis_name="c", subcore_axis_name="s")

_sc_cp = pltpu.CompilerParams()
if "needs_layout_passes" in pltpu.CompilerParams.__dataclass_fields__:
    _sc_cp = dataclasses.replace(_sc_cp, needs_layout_passes=False)


@functools.partial(
    pl.kernel,
    mesh=_mesh,
    compiler_params=_sc_cp,
    out_type=_sds((_NC * N, H), _F32),
    scratch_types=[
        pltpu.VMEM((_BC,), jnp.int32),
        pltpu.VMEM((_BC,), jnp.int32),
        pltpu.VMEM((_BC, H), _F32),
        pltpu.VMEM((_BC, H), _F32),
        pltpu.VMEM((_BC, H), _F32),
        pltpu.VMEM((_ZR, H), _F32),
        pltpu.VMEM_SHARED((N, H), _F32),
        pltpu.SemaphoreType.DMA,
    ],
)
def _sc_edge(h3_hbm, h4_hbm, row_hbm, col_hbm, out_hbm,
             rowi, coli, h3r, h4r, h4c, zbuf, acc, sem):
    ci = lax.axis_index("c")
    si = lax.axis_index("s")

    @pl.loop(0, _ZR)
    def _(i):
        for j in range(_LC):
            zbuf[i, pl.ds(j * 16, 16)] = jnp.zeros((16,), _F32)

    @pl.loop(0, _DR // _ZR)
    def _(t):
        off = pl.multiple_of(si * _DR + t * _ZR, 8)
        pltpu.sync_copy(zbuf, acc.at[pl.ds(off, _ZR)])

    @pl.when(si == _NS - 1)
    def _():
        pltpu.sync_copy(zbuf.at[pl.ds(0, _TAIL)],
                        acc.at[pl.ds(_NS * _DR, _TAIL)])

    plsc.subcore_barrier()

    @pl.loop(0, _NCH)
    def _(c):
        eb = (ci * _NS + si) * _EPW + c * _BC
        pltpu.sync_copy(row_hbm.at[pl.ds(eb, _BC)], rowi)
        pltpu.sync_copy(col_hbm.at[pl.ds(eb, _BC)], coli)
        g1 = pltpu.async_copy(h3_hbm.at[rowi], h3r, sem)
        g2 = pltpu.async_copy(h4_hbm.at[rowi], h4r, sem)
        g3 = pltpu.async_copy(h4_hbm.at[coli], h4c, sem)
        g1.wait()
        g2.wait()
        g3.wait()

        @plsc.parallel_loop(0, _BC, unroll=4)
        def _(e):
            h3v = [h3r[e, pl.ds(16 * j, 16)] for j in range(_LC)]
            h4cv = [h4c[e, pl.ds(16 * j, 16)] for j in range(_LC)]
            accv = h3v[0] * h4cv[0]
            for j in range(1, _LC):
                accv = accv + h3v[j] * h4cv[j]
            dvec = jnp.full((16,), jnp.sum(accv), _F32)
            svec = 1.0 / (1.0 + jnp.exp(dvec))
            for j in range(_LC):
                h4v = h4r[e, pl.ds(16 * j, 16)]
                h4c[e, pl.ds(16 * j, 16)] = svec * (h3v[j] + h4v) - h4v

        pltpu.sync_copy(h4c, acc.at[coli], add=True)

    plsc.subcore_barrier()

    doff = pl.multiple_of(si * _DR, 8)
    ooff = pl.multiple_of(ci * N + si * _DR, 8)
    pltpu.sync_copy(acc.at[pl.ds(doff, _DR)], out_hbm.at[pl.ds(ooff, _DR)])

    @pl.when(si == _NS - 1)
    def _():
        toff = pl.multiple_of(ci * N + _NS * _DR, 8)
        pltpu.sync_copy(acc.at[pl.ds(_NS * _DR, _TAIL)],
                        out_hbm.at[pl.ds(toff, _TAIL)])


def kernel(x, edge_index, adj, ADJ1, W_in, b_in, W1, W2, Wf, W_out, b_out):
    row = edge_index[0].astype(jnp.int32)
    col = edge_index[1].astype(jnp.int32)
    b_in2 = b_in.reshape(1, H)
    b_out2 = b_out.reshape(1, C)

    h3, h4, h1 = _tc0(x, W_in, b_in2, W1[0], W2[0], Wf[0])
    p = _sc_edge(h3, h4, row, col)
    h3, h4, h1 = _tc1(h1, p[:N], p[N:], W1[1], W2[1], Wf[1])
    p = _sc_edge(h3, h4, row, col)
    return _tc2(h1, p[:N], p[N:], W_out, b_out2)
